# Initial kernel scaffold; baseline (speedup 1.0000x reference)
import jax
import jax.numpy as jnp
from jax.experimental import pallas as pl


def kernel(x, edge_index, edge_attr, node_areas, W1, b1, W2, b2):
    raise NotImplementedError("write your pallas kernel here")


# speedup vs baseline: 17.9536x; 17.9536x over previous
import functools

import jax
import jax.numpy as jnp
from jax import lax
from jax.experimental import pallas as pl
from jax.experimental.pallas import tpu as pltpu
from jax.experimental.pallas import tpu_sc as plsc

_NC = 2
_NS = 16
_L = 16


def _row_mean(x):
    n, d = x.shape

    def mk(x_ref, o_ref):
        o_ref[...] = jnp.mean(x_ref[...], axis=1, keepdims=True)

    return pl.pallas_call(
        mk,
        out_shape=jax.ShapeDtypeStruct((n, 1), jnp.float32),
    )(x)


def _sc_features(m, areas, src, dst, ea_flat):
    e = src.shape[0]
    n = m.shape[0]
    nw = _NC * _NS
    n_per = e // nw
    ch = 2000
    n_chunks = n_per // ch
    assert n_per * nw == e and n_chunks * ch == n_per

    mesh = plsc.VectorSubcoreMesh(core_axis_name="c", subcore_axis_name="s")

    @functools.partial(
        pl.kernel,
        mesh=mesh,
        compiler_params=pltpu.CompilerParams(needs_layout_passes=False),
        out_type=jax.ShapeDtypeStruct((4 * e,), jnp.float32),
        scratch_types=[
            pltpu.VMEM((n,), jnp.float32),
            pltpu.VMEM((n,), jnp.float32),
            pltpu.VMEM((ch,), jnp.int32),
            pltpu.VMEM((ch,), jnp.int32),
            pltpu.VMEM((4 * ch,), jnp.float32),
            pltpu.VMEM((ch,), jnp.float32),
            pltpu.VMEM((ch,), jnp.float32),
            pltpu.VMEM((ch,), jnp.float32),
            pltpu.VMEM((ch,), jnp.float32),
        ],
    )
    def k(m_hbm, a_hbm, src_hbm, dst_hbm, ea_hbm, f_hbm,
          m_v, a_v, src_v, dst_v, ea_v, hd_v, sl_v, ln_v, aa_v):
        wid = lax.axis_index("s") * _NC + lax.axis_index("c")
        base = wid * n_per
        pltpu.sync_copy(m_hbm, m_v)
        pltpu.sync_copy(a_hbm, a_v)
        iota = lax.iota(jnp.int32, _L)
        iota4 = iota * 4

        def chunk_body(c, _):
            cb = pl.multiple_of(base + c * ch, 8)
            pltpu.sync_copy(src_hbm.at[pl.ds(cb, ch)], src_v)
            pltpu.sync_copy(dst_hbm.at[pl.ds(cb, ch)], dst_v)
            pltpu.sync_copy(ea_hbm.at[pl.ds(cb * 4, ch * 4)], ea_v)

            def vec_body(i, _):
                o = pl.multiple_of(i * _L, _L)
                s = src_v[pl.ds(o, _L)]
                d = dst_v[pl.ds(o, _L)]
                ms = plsc.load_gather(m_v, [s])
                md = plsc.load_gather(m_v, [d])
                sa = plsc.load_gather(a_v, [s])
                da = plsc.load_gather(a_v, [d])
                eidx = iota4 + o * 4
                sl = plsc.load_gather(ea_v, [eidx])
                ln = plsc.load_gather(ea_v, [eidx + 1])
                hd_v[pl.ds(o, _L)] = ms - md
                sl_v[pl.ds(o, _L)] = sl
                ln_v[pl.ds(o, _L)] = ln
                aa_v[pl.ds(o, _L)] = 2.0 * sa * da / (sa + da + 1e-8)
                return 0

            lax.fori_loop(0, ch // _L, vec_body, 0)
            pltpu.sync_copy(hd_v, f_hbm.at[pl.ds(cb, ch)])
            pltpu.sync_copy(sl_v, f_hbm.at[pl.ds(e + cb, ch)])
            pltpu.sync_copy(ln_v, f_hbm.at[pl.ds(2 * e + cb, ch)])
            pltpu.sync_copy(aa_v, f_hbm.at[pl.ds(3 * e + cb, ch)])
            return 0

        lax.fori_loop(0, n_chunks, chunk_body, 0)

    return k(m, areas, src, dst, ea_flat)


def _mlp(f, w1t, b1c, w2c, b2c):
    e = f.shape[1]
    blk = 2560
    grid = e // blk
    assert grid * blk == e

    def mk(f_ref, w1t_ref, b1_ref, w2_ref, b2_ref, o_ref):
        fb = f_ref[...]
        w1t = w1t_ref[...]
        h = b1_ref[...]
        h = (h
             + w1t[:, 0:1] * fb[0:1, :]
             + w1t[:, 1:2] * fb[1:2, :]
             + w1t[:, 2:3] * fb[2:3, :]
             + w1t[:, 3:4] * fb[3:4, :])
        h = jnp.maximum(h, 0.0)
        o_ref[...] = jnp.sum(h * w2_ref[...], axis=0, keepdims=True) + b2_ref[...]

    return pl.pallas_call(
        mk,
        grid=(grid,),
        in_specs=[
            pl.BlockSpec((4, blk), lambda i: (0, i)),
            pl.BlockSpec((32, 4), lambda i: (0, 0)),
            pl.BlockSpec((32, 1), lambda i: (0, 0)),
            pl.BlockSpec((32, 1), lambda i: (0, 0)),
            pl.BlockSpec((1, 1), lambda i: (0, 0)),
        ],
        out_specs=pl.BlockSpec((1, blk), lambda i: (0, i)),
        out_shape=jax.ShapeDtypeStruct((1, e), jnp.float32),
    )(f, w1t, b1c, w2c, b2c)


def kernel(x, edge_index, edge_attr, node_areas, W1, b1, W2, b2):
    e = edge_index.shape[1]
    src = edge_index[0].astype(jnp.int32)
    dst = edge_index[1].astype(jnp.int32)
    m = _row_mean(x.astype(jnp.float32)).reshape(-1)
    f_flat = _sc_features(m, node_areas.astype(jnp.float32), src, dst,
                          edge_attr.astype(jnp.float32).reshape(-1))
    f = f_flat.reshape(4, e)
    out = _mlp(f, W1.T.astype(jnp.float32), b1.reshape(-1, 1),
               W2.reshape(-1, 1), b2.reshape(1, 1))
    return out.reshape(e, 1)


# speedup vs baseline: 65.5598x; 3.6516x over previous
import functools

import jax
import jax.numpy as jnp
from jax import lax
from jax.experimental import pallas as pl
from jax.experimental.pallas import tpu as pltpu
from jax.experimental.pallas import tpu_sc as plsc

_NC = 2
_NS = 16
_L = 16
_UNROLL = 5


def _row_mean(x):
    n, d = x.shape

    def mk(x_ref, o_ref):
        o_ref[...] = jnp.mean(x_ref[...], axis=1)

    return pl.pallas_call(
        mk,
        out_shape=jax.ShapeDtypeStruct((n,), jnp.float32),
    )(x)


def _sc_hd_aa(m, areas, src, dst):
    e = src.shape[0]
    n = m.shape[0]
    nw = _NC * _NS
    n_per = e // nw
    ch = 2000
    n_chunks = n_per // ch
    assert n_per * nw == e and n_chunks * ch == n_per
    assert ch % (_L * _UNROLL) == 0

    mesh = plsc.VectorSubcoreMesh(core_axis_name="c", subcore_axis_name="s")

    @functools.partial(
        pl.kernel,
        mesh=mesh,
        compiler_params=pltpu.CompilerParams(needs_layout_passes=False),
        out_type=(
            jax.ShapeDtypeStruct((e,), jnp.float32),
            jax.ShapeDtypeStruct((e,), jnp.float32),
        ),
        scratch_types=[
            pltpu.VMEM((n,), jnp.float32),
            pltpu.VMEM((n,), jnp.float32),
            pltpu.VMEM((ch,), jnp.int32),
            pltpu.VMEM((ch,), jnp.int32),
            pltpu.VMEM((ch,), jnp.float32),
            pltpu.VMEM((ch,), jnp.float32),
        ],
    )
    def k(m_hbm, a_hbm, src_hbm, dst_hbm, hd_hbm, aa_hbm,
          m_v, a_v, src_v, dst_v, hd_v, aa_v):
        wid = lax.axis_index("s") * _NC + lax.axis_index("c")
        base = wid * n_per
        pltpu.sync_copy(m_hbm, m_v)
        pltpu.sync_copy(a_hbm, a_v)

        def chunk_body(c, _):
            cb = pl.multiple_of(base + c * ch, 8)
            pltpu.sync_copy(src_hbm.at[pl.ds(cb, ch)], src_v)
            pltpu.sync_copy(dst_hbm.at[pl.ds(cb, ch)], dst_v)

            def vec_body(i, _):
                o0 = pl.multiple_of(i * (_L * _UNROLL), _L * _UNROLL)
                for u in range(_UNROLL):
                    o = o0 + u * _L
                    s = src_v[pl.ds(o, _L)]
                    d = dst_v[pl.ds(o, _L)]
                    ms = plsc.load_gather(m_v, [s])
                    md = plsc.load_gather(m_v, [d])
                    sa = plsc.load_gather(a_v, [s])
                    da = plsc.load_gather(a_v, [d])
                    hd_v[pl.ds(o, _L)] = ms - md
                    aa_v[pl.ds(o, _L)] = 2.0 * sa * da / (sa + da + 1e-8)
                return 0

            lax.fori_loop(0, ch // (_L * _UNROLL), vec_body, 0)
            pltpu.sync_copy(hd_v, hd_hbm.at[pl.ds(cb, ch)])
            pltpu.sync_copy(aa_v, aa_hbm.at[pl.ds(cb, ch)])
            return 0

        lax.fori_loop(0, n_chunks, chunk_body, 0)

    return k(m, areas, src, dst)


def _mlp(hd, aa, ea_t, w1t, b1c, w2t, b2c):
    e = hd.shape[0]
    blk = 8192
    grid = -(-e // blk)

    def mk(hd_ref, aa_ref, ea_ref, w1t_ref, b1_ref, w2t_ref, b2_ref, o_ref):
        hdb = hd_ref[...].reshape(1, blk)
        aab = aa_ref[...].reshape(1, blk)
        eab = ea_ref[...]
        fb = jnp.concatenate([hdb, eab[0:1, :], eab[1:2, :], aab], axis=0)
        h = jnp.dot(w1t_ref[...], fb, preferred_element_type=jnp.float32)
        h = jnp.maximum(h + b1_ref[...], 0.0)
        o = jnp.dot(w2t_ref[...], h, preferred_element_type=jnp.float32)
        o_ref[...] = (o + b2_ref[...]).reshape(blk)

    return pl.pallas_call(
        mk,
        grid=(grid,),
        in_specs=[
            pl.BlockSpec((blk,), lambda i: (i,)),
            pl.BlockSpec((blk,), lambda i: (i,)),
            pl.BlockSpec((4, blk), lambda i: (0, i)),
            pl.BlockSpec((32, 4), lambda i: (0, 0)),
            pl.BlockSpec((32, 1), lambda i: (0, 0)),
            pl.BlockSpec((1, 32), lambda i: (0, 0)),
            pl.BlockSpec((1, 1), lambda i: (0, 0)),
        ],
        out_specs=pl.BlockSpec((blk,), lambda i: (i,)),
        out_shape=jax.ShapeDtypeStruct((e,), jnp.float32),
    )(hd, aa, ea_t, w1t, b1c, w2t, b2c)


def kernel(x, edge_index, edge_attr, node_areas, W1, b1, W2, b2):
    e = edge_index.shape[1]
    src = edge_index[0].astype(jnp.int32)
    dst = edge_index[1].astype(jnp.int32)
    m = _row_mean(x.astype(jnp.float32))
    hd, aa = _sc_hd_aa(m, node_areas.astype(jnp.float32), src, dst)
    out = _mlp(hd, aa, edge_attr.astype(jnp.float32).T,
               W1.astype(jnp.float32).T, b1.astype(jnp.float32).reshape(-1, 1),
               W2.astype(jnp.float32).reshape(1, -1),
               b2.astype(jnp.float32).reshape(1, 1))
    return out.reshape(e, 1)


# speedup vs baseline: 75.2180x; 1.1473x over previous
import functools

import jax
import jax.numpy as jnp
from jax import lax
from jax.experimental import pallas as pl
from jax.experimental.pallas import tpu as pltpu
from jax.experimental.pallas import tpu_sc as plsc

_NC = 2
_NS = 16
_L = 16
_UNROLL = 5


def _row_mean(x):
    n, d = x.shape

    def mk(x_ref, o_ref):
        o_ref[...] = jnp.mean(x_ref[...], axis=1)

    return pl.pallas_call(
        mk,
        out_shape=jax.ShapeDtypeStruct((n,), jnp.float32),
    )(x)


def _sc_hd_aa(m, areas, src, dst):
    e = src.shape[0]
    n = m.shape[0]
    nw = _NC * _NS
    n_per = e // nw
    ch = 2000
    n_chunks = n_per // ch
    assert n_per * nw == e and n_chunks * ch == n_per
    assert ch % (_L * _UNROLL) == 0

    mesh = plsc.VectorSubcoreMesh(core_axis_name="c", subcore_axis_name="s")

    @functools.partial(
        pl.kernel,
        mesh=mesh,
        compiler_params=pltpu.CompilerParams(needs_layout_passes=False),
        out_type=(
            jax.ShapeDtypeStruct((e,), jnp.float32),
            jax.ShapeDtypeStruct((e,), jnp.float32),
        ),
        scratch_types=[
            pltpu.VMEM((n,), jnp.float32),
            pltpu.VMEM((n,), jnp.float32),
            pltpu.VMEM((ch,), jnp.int32),
            pltpu.VMEM((ch,), jnp.int32),
            pltpu.VMEM((ch,), jnp.int32),
            pltpu.VMEM((ch,), jnp.int32),
            pltpu.VMEM((ch,), jnp.float32),
            pltpu.VMEM((ch,), jnp.float32),
            pltpu.VMEM((ch,), jnp.float32),
            pltpu.VMEM((ch,), jnp.float32),
            pltpu.SemaphoreType.DMA,
            pltpu.SemaphoreType.DMA,
            pltpu.SemaphoreType.DMA,
            pltpu.SemaphoreType.DMA,
        ],
    )
    def k(m_hbm, a_hbm, src_hbm, dst_hbm, hd_hbm, aa_hbm,
          m_v, a_v, s0, s1, d0, d1, h0, h1, a0, a1,
          sin0, sin1, sout0, sout1):
        wid = lax.axis_index("s") * _NC + lax.axis_index("c")
        base = wid * n_per
        pltpu.sync_copy(m_hbm, m_v)
        pltpu.sync_copy(a_hbm, a_v)
        sbufs, dbufs = (s0, s1), (d0, d1)
        hbufs, abufs = (h0, h1), (a0, a1)
        sins, souts = (sin0, sin1), (sout0, sout1)

        def cb_of(c):
            return pl.multiple_of(base + c * ch, 8)

        def start_in(c):
            b = c % 2
            pltpu.async_copy(src_hbm.at[pl.ds(cb_of(c), ch)], sbufs[b], sins[b])
            pltpu.async_copy(dst_hbm.at[pl.ds(cb_of(c), ch)], dbufs[b], sins[b])

        start_in(0)
        out_handles = [None, None]
        for c in range(n_chunks):
            b = c % 2
            if c + 1 < n_chunks:
                start_in(c + 1)
            pltpu.make_async_copy(
                src_hbm.at[pl.ds(cb_of(c), ch)], sbufs[b], sins[b]).wait()
            pltpu.make_async_copy(
                dst_hbm.at[pl.ds(cb_of(c), ch)], dbufs[b], sins[b]).wait()
            if out_handles[b] is not None:
                for hnd in out_handles[b]:
                    hnd.wait()
            src_v, dst_v = sbufs[b], dbufs[b]
            hd_v, aa_v = hbufs[b], abufs[b]

            @plsc.parallel_loop(0, ch, step=_L, unroll=_UNROLL)
            def vec_body(o):
                s = src_v[pl.ds(o, _L)]
                d = dst_v[pl.ds(o, _L)]
                ms = plsc.load_gather(m_v, [s])
                md = plsc.load_gather(m_v, [d])
                sa = plsc.load_gather(a_v, [s])
                da = plsc.load_gather(a_v, [d])
                hd_v[pl.ds(o, _L)] = ms - md
                aa_v[pl.ds(o, _L)] = 2.0 * sa * da / (sa + da + 1e-8)

            out_handles[b] = (
                pltpu.async_copy(hd_v, hd_hbm.at[pl.ds(cb_of(c), ch)], souts[b]),
                pltpu.async_copy(aa_v, aa_hbm.at[pl.ds(cb_of(c), ch)], souts[b]),
            )
        for hs in out_handles:
            if hs is not None:
                for hnd in hs:
                    hnd.wait()

    return k(m, areas, src, dst)


def _mlp(hd, aa, ea_t, w1t, b1c, w2t, b2c):
    e = hd.shape[0]
    blk = 8192
    grid = -(-e // blk)

    def mk(hd_ref, aa_ref, ea_ref, w1t_ref, b1_ref, w2t_ref, b2_ref, o_ref):
        hdb = hd_ref[...].reshape(1, blk)
        aab = aa_ref[...].reshape(1, blk)
        eab = ea_ref[...]
        fb = jnp.concatenate([hdb, eab[0:1, :], eab[1:2, :], aab], axis=0)
        h = jnp.dot(w1t_ref[...], fb, preferred_element_type=jnp.float32)
        h = jnp.maximum(h + b1_ref[...], 0.0)
        o = jnp.dot(w2t_ref[...], h, preferred_element_type=jnp.float32)
        o_ref[...] = (o + b2_ref[...]).reshape(blk)

    return pl.pallas_call(
        mk,
        grid=(grid,),
        in_specs=[
            pl.BlockSpec((blk,), lambda i: (i,)),
            pl.BlockSpec((blk,), lambda i: (i,)),
            pl.BlockSpec((4, blk), lambda i: (0, i)),
            pl.BlockSpec((32, 4), lambda i: (0, 0)),
            pl.BlockSpec((32, 1), lambda i: (0, 0)),
            pl.BlockSpec((1, 32), lambda i: (0, 0)),
            pl.BlockSpec((1, 1), lambda i: (0, 0)),
        ],
        out_specs=pl.BlockSpec((blk,), lambda i: (i,)),
        out_shape=jax.ShapeDtypeStruct((e,), jnp.float32),
    )(hd, aa, ea_t, w1t, b1c, w2t, b2c)


def kernel(x, edge_index, edge_attr, node_areas, W1, b1, W2, b2):
    e = edge_index.shape[1]
    src = edge_index[0].astype(jnp.int32)
    dst = edge_index[1].astype(jnp.int32)
    m = _row_mean(x.astype(jnp.float32))
    hd, aa = _sc_hd_aa(m, node_areas.astype(jnp.float32), src, dst)
    out = _mlp(hd, aa, edge_attr.astype(jnp.float32).T,
               W1.astype(jnp.float32).T, b1.astype(jnp.float32).reshape(-1, 1),
               W2.astype(jnp.float32).reshape(1, -1),
               b2.astype(jnp.float32).reshape(1, 1))
    return out.reshape(e, 1)


# speedup vs baseline: 87.4969x; 1.1632x over previous
import functools

import jax
import jax.numpy as jnp
from jax import lax
from jax.experimental import pallas as pl
from jax.experimental.pallas import tpu as pltpu
from jax.experimental.pallas import tpu_sc as plsc

_NC = 2
_NS = 16
_L = 16
_UNROLL = 5


def _split_edge_index(ei):
    e = ei.shape[1]
    blk = 8192
    grid = -(-e // blk)

    def mk(ei_ref, s_ref, d_ref):
        b = ei_ref[...]
        s_ref[...] = b[0:1, :].reshape(blk)
        d_ref[...] = b[1:2, :].reshape(blk)

    return pl.pallas_call(
        mk,
        grid=(grid,),
        in_specs=[pl.BlockSpec((2, blk), lambda i: (0, i))],
        out_specs=(pl.BlockSpec((blk,), lambda i: (i,)),
                   pl.BlockSpec((blk,), lambda i: (i,))),
        out_shape=(jax.ShapeDtypeStruct((e,), jnp.int32),
                   jax.ShapeDtypeStruct((e,), jnp.int32)),
    )(ei)


def _row_mean(x):
    n, d = x.shape

    def mk(x_ref, o_ref):
        o_ref[...] = jnp.mean(x_ref[...], axis=1)

    return pl.pallas_call(
        mk,
        out_shape=jax.ShapeDtypeStruct((n,), jnp.float32),
    )(x)


def _sc_hd_aa(m, areas, src, dst):
    e = src.shape[0]
    n = m.shape[0]
    nw = _NC * _NS
    n_per = e // nw
    ch = 2000
    n_chunks = n_per // ch
    assert n_per * nw == e and n_chunks * ch == n_per
    assert ch % (_L * _UNROLL) == 0

    mesh = plsc.VectorSubcoreMesh(core_axis_name="c", subcore_axis_name="s")

    @functools.partial(
        pl.kernel,
        mesh=mesh,
        compiler_params=pltpu.CompilerParams(needs_layout_passes=False),
        out_type=(
            jax.ShapeDtypeStruct((e,), jnp.float32),
            jax.ShapeDtypeStruct((e,), jnp.float32),
        ),
        scratch_types=[
            pltpu.VMEM((n,), jnp.float32),
            pltpu.VMEM((n,), jnp.float32),
            pltpu.VMEM((ch,), jnp.int32),
            pltpu.VMEM((ch,), jnp.int32),
            pltpu.VMEM((ch,), jnp.int32),
            pltpu.VMEM((ch,), jnp.int32),
            pltpu.VMEM((ch,), jnp.float32),
            pltpu.VMEM((ch,), jnp.float32),
            pltpu.VMEM((ch,), jnp.float32),
            pltpu.VMEM((ch,), jnp.float32),
            pltpu.SemaphoreType.DMA,
            pltpu.SemaphoreType.DMA,
            pltpu.SemaphoreType.DMA,
            pltpu.SemaphoreType.DMA,
        ],
    )
    def k(m_hbm, a_hbm, src_hbm, dst_hbm, hd_hbm, aa_hbm,
          m_v, a_v, s0, s1, d0, d1, h0, h1, a0, a1,
          sin0, sin1, sout0, sout1):
        wid = lax.axis_index("s") * _NC + lax.axis_index("c")
        base = wid * n_per
        pltpu.sync_copy(m_hbm, m_v)
        pltpu.sync_copy(a_hbm, a_v)
        sbufs, dbufs = (s0, s1), (d0, d1)
        hbufs, abufs = (h0, h1), (a0, a1)
        sins, souts = (sin0, sin1), (sout0, sout1)

        def cb_of(c):
            return pl.multiple_of(base + c * ch, 8)

        def start_in(c):
            b = c % 2
            pltpu.async_copy(src_hbm.at[pl.ds(cb_of(c), ch)], sbufs[b], sins[b])
            pltpu.async_copy(dst_hbm.at[pl.ds(cb_of(c), ch)], dbufs[b], sins[b])

        start_in(0)
        out_handles = [None, None]
        for c in range(n_chunks):
            b = c % 2
            if c + 1 < n_chunks:
                start_in(c + 1)
            pltpu.make_async_copy(
                src_hbm.at[pl.ds(cb_of(c), ch)], sbufs[b], sins[b]).wait()
            pltpu.make_async_copy(
                dst_hbm.at[pl.ds(cb_of(c), ch)], dbufs[b], sins[b]).wait()
            if out_handles[b] is not None:
                for hnd in out_handles[b]:
                    hnd.wait()
            src_v, dst_v = sbufs[b], dbufs[b]
            hd_v, aa_v = hbufs[b], abufs[b]

            @plsc.parallel_loop(0, ch, step=_L, unroll=_UNROLL)
            def vec_body(o):
                s = src_v[pl.ds(o, _L)]
                d = dst_v[pl.ds(o, _L)]
                ms = plsc.load_gather(m_v, [s])
                md = plsc.load_gather(m_v, [d])
                sa = plsc.load_gather(a_v, [s])
                da = plsc.load_gather(a_v, [d])
                hd_v[pl.ds(o, _L)] = ms - md
                aa_v[pl.ds(o, _L)] = 2.0 * sa * da / (sa + da + 1e-8)

            out_handles[b] = (
                pltpu.async_copy(hd_v, hd_hbm.at[pl.ds(cb_of(c), ch)], souts[b]),
                pltpu.async_copy(aa_v, aa_hbm.at[pl.ds(cb_of(c), ch)], souts[b]),
            )
        for hs in out_handles:
            if hs is not None:
                for hnd in hs:
                    hnd.wait()

    return k(m, areas, src, dst)


def _mlp(hd, aa, ea_t, w1t, b1c, w2t, b2c):
    e = hd.shape[0]
    blk = 16384
    grid = -(-e // blk)

    def mk(hd_ref, aa_ref, ea_ref, w1t_ref, b1_ref, w2t_ref, b2_ref, o_ref):
        hdb = hd_ref[...].reshape(1, blk)
        aab = aa_ref[...].reshape(1, blk)
        eab = ea_ref[...]
        fb = jnp.concatenate([hdb, eab[0:1, :], eab[1:2, :], aab], axis=0)
        h = jnp.dot(w1t_ref[...], fb, preferred_element_type=jnp.float32)
        h = jnp.maximum(h + b1_ref[...], 0.0)
        o = jnp.dot(w2t_ref[...], h, preferred_element_type=jnp.float32)
        o_ref[...] = o + b2_ref[...]

    return pl.pallas_call(
        mk,
        grid=(grid,),
        in_specs=[
            pl.BlockSpec((blk,), lambda i: (i,)),
            pl.BlockSpec((blk,), lambda i: (i,)),
            pl.BlockSpec((4, blk), lambda i: (0, i)),
            pl.BlockSpec((32, 4), lambda i: (0, 0)),
            pl.BlockSpec((32, 1), lambda i: (0, 0)),
            pl.BlockSpec((1, 32), lambda i: (0, 0)),
            pl.BlockSpec((1, 1), lambda i: (0, 0)),
        ],
        out_specs=pl.BlockSpec((1, blk), lambda i: (0, i)),
        out_shape=jax.ShapeDtypeStruct((1, e), jnp.float32),
    )(hd, aa, ea_t, w1t, b1c, w2t, b2c)


def kernel(x, edge_index, edge_attr, node_areas, W1, b1, W2, b2):
    e = edge_index.shape[1]
    src, dst = _split_edge_index(edge_index.astype(jnp.int32))
    m = _row_mean(x.astype(jnp.float32))
    hd, aa = _sc_hd_aa(m, node_areas.astype(jnp.float32), src, dst)
    out = _mlp(hd, aa, edge_attr.astype(jnp.float32).T,
               W1.astype(jnp.float32).T, b1.astype(jnp.float32).reshape(-1, 1),
               W2.astype(jnp.float32).reshape(1, -1),
               b2.astype(jnp.float32).reshape(1, 1))
    return out.reshape(e, 1)
